# Initial kernel scaffold; baseline (speedup 1.0000x reference)
import jax
import jax.numpy as jnp
from jax.experimental import pallas as pl


def kernel(x, h, edge_index, mask_nodes, W_dec, b_dec, inference):
    raise NotImplementedError("write your pallas kernel here")


# speedup vs baseline: 5.1223x; 5.1223x over previous
import functools
import jax
import jax.numpy as jnp
from jax import lax
from jax.experimental import pallas as pl
from jax.experimental.pallas import tpu as pltpu
from jax.experimental.pallas import tpu_sc as plsc

NC = 2
NS = 16
NW = NC * NS
G = 80
L = 16


def _mm_body(h_ref, w_ref, o_ref):
    o_ref[...] = jnp.dot(h_ref[...], w_ref[...],
                         preferred_element_type=jnp.float32)


def _matmul(h, w):
    n, d = h.shape
    blk = 1000
    return pl.pallas_call(
        _mm_body,
        grid=(n // blk,),
        in_specs=[
            pl.BlockSpec((blk, d), lambda i: (i, 0)),
            pl.BlockSpec((d, d), lambda i: (0, 0)),
        ],
        out_specs=pl.BlockSpec((blk, d), lambda i: (i, 0)),
        out_shape=jax.ShapeDtypeStruct((n, d), jnp.float32),
    )(h, w)


def _make_sc_segsum(n_pad, d, e):
    ec = e // NW
    nb = ec // G
    rows_pt = n_pad // NS
    mesh = plsc.VectorSubcoreMesh(core_axis_name="c", subcore_axis_name="s")

    @functools.partial(
        pl.kernel,
        out_type=(
            jax.ShapeDtypeStruct((n_pad, d), jnp.float32),
            jax.ShapeDtypeStruct((n_pad, d), jnp.float32),
        ),
        mesh=mesh,
        scratch_types=[
            pltpu.VMEM((G,), jnp.int32),
            pltpu.VMEM((G,), jnp.int32),
            pltpu.VMEM((G, d), jnp.float32),
            pltpu.VMEM_SHARED((n_pad, d), jnp.float32),
            pltpu.SemaphoreType.DMA,
        ],
    )
    def sc_segsum(src_hbm, dst_hbm, hw_hbm, zrows_hbm, p0_hbm, p1_hbm,
                  idxs_v, idxd_v, rows_v, acc_sh, sem):
        cid = lax.axis_index("c")
        sid = lax.axis_index("s")
        wid = cid * NS + sid
        base = wid * ec

        pltpu.sync_copy(zrows_hbm, acc_sh.at[pl.ds(sid * rows_pt, rows_pt)])
        plsc.subcore_barrier()

        def blk(b, carry):
            off = base + b * G
            pltpu.sync_copy(src_hbm.at[pl.ds(off, G)], idxs_v)
            pltpu.sync_copy(dst_hbm.at[pl.ds(off, G)], idxd_v)
            pltpu.async_copy(hw_hbm.at[idxs_v], rows_v, sem).wait()
            pltpu.sync_copy(rows_v, acc_sh.at[idxd_v], add=True)
            return carry
        lax.fori_loop(0, nb, blk, 0)
        plsc.subcore_barrier()

        @pl.when(cid == 0)
        def _():
            pltpu.sync_copy(acc_sh.at[pl.ds(sid * rows_pt, rows_pt)],
                            p0_hbm.at[pl.ds(sid * rows_pt, rows_pt)])

        @pl.when(cid == 1)
        def _():
            pltpu.sync_copy(acc_sh.at[pl.ds(sid * rows_pt, rows_pt)],
                            p1_hbm.at[pl.ds(sid * rows_pt, rows_pt)])

    return sc_segsum


def _make_sc_loss(n_pad, d, m_pad, m_real):
    mc = m_pad // NW
    mesh = plsc.VectorSubcoreMesh(core_axis_name="c", subcore_axis_name="s")

    @functools.partial(
        pl.kernel,
        out_type=jax.ShapeDtypeStruct((NW * L,), jnp.float32),
        mesh=mesh,
        scratch_types=[
            pltpu.VMEM((G,), jnp.int32),
            pltpu.VMEM((G, d), jnp.float32),
            pltpu.VMEM((G, d), jnp.float32),
            pltpu.VMEM((G, d), jnp.float32),
            pltpu.VMEM((d,), jnp.float32),
            pltpu.VMEM((L,), jnp.float32),
            pltpu.SemaphoreType.DMA,
        ],
    )
    def sc_loss(p0_hbm, p1_hbm, x_hbm, mi_hbm, b_hbm, out_hbm,
                mi_v, r0_v, r1_v, rx_v, b_v, acc_v, sem):
        cid = lax.axis_index("c")
        sid = lax.axis_index("s")
        wid = cid * NS + sid

        pltpu.sync_copy(b_hbm, b_v)
        acc_v[...] = jnp.zeros((L,), jnp.float32)

        for j in range(mc // G):
            off = wid * mc + j * G
            pltpu.sync_copy(mi_hbm.at[pl.ds(off, G)], mi_v)
            pltpu.async_copy(p0_hbm.at[mi_v], r0_v, sem).wait()
            pltpu.async_copy(p1_hbm.at[mi_v], r1_v, sem).wait()
            pltpu.async_copy(x_hbm.at[mi_v], rx_v, sem).wait()

            def row(r, carry):
                vs = jnp.where(off + r < m_real, jnp.float32(1.0),
                               jnp.float32(0.0))
                valid = jnp.full((L,), vs)
                s = jnp.zeros((L,), jnp.float32)
                for c in range(d // L):
                    cs = pl.ds(c * L, L)
                    dv = (r0_v[r, cs] + r1_v[r, cs] + b_v[cs]) - rx_v[r, cs]
                    s = s + dv * dv
                acc_v[...] = acc_v[...] + valid * s
                return carry
            lax.fori_loop(0, G, row, 0)

        pltpu.sync_copy(acc_v, out_hbm.at[pl.ds(wid * L, L)])

    return sc_loss


def kernel(x, h, edge_index, mask_nodes, W_dec, b_dec, inference=False):
    n, d = x.shape
    e = edge_index.shape[1]
    m = mask_nodes.shape[0]
    mc = -(-m // NW)
    mc = -(-mc // G) * G
    m_pad = mc * NW
    n_pad = -(-n // (NS * 8)) * (NS * 8)

    src = edge_index[0].astype(jnp.int32)
    dst = edge_index[1].astype(jnp.int32)
    mi = jnp.zeros((m_pad,), jnp.int32).at[:m].set(mask_nodes.astype(jnp.int32))
    zrows = jnp.zeros((n_pad // NS, d), jnp.float32)

    hw = _matmul(h, W_dec)
    p0, p1 = _make_sc_segsum(n_pad, d, e)(src, dst, hw, zrows)
    lane_sums = _make_sc_loss(n_pad, d, m_pad, m)(p0, p1, x, mi, b_dec)
    loss = jnp.sum(lane_sums) / jnp.float32(m * d)
    return jnp.where(inference, jnp.float32(0.0), loss)


# speedup vs baseline: 8.5114x; 1.6617x over previous
import functools
import jax
import jax.numpy as jnp
from jax import lax
from jax.experimental import pallas as pl
from jax.experimental.pallas import tpu as pltpu
from jax.experimental.pallas import tpu_sc as plsc

NC = 2
NS = 16
NW = NC * NS
G = 80
L = 16


def _mm_body(h_ref, w_ref, o_ref):
    o_ref[...] = jnp.dot(h_ref[...], w_ref[...],
                         preferred_element_type=jnp.float32)


def _matmul(h, w):
    n, d = h.shape
    blk = 1000
    return pl.pallas_call(
        _mm_body,
        grid=(n // blk,),
        in_specs=[
            pl.BlockSpec((blk, d), lambda i: (i, 0)),
            pl.BlockSpec((d, d), lambda i: (0, 0)),
        ],
        out_specs=pl.BlockSpec((blk, d), lambda i: (i, 0)),
        out_shape=jax.ShapeDtypeStruct((n, d), jnp.float32),
    )(h, w)


def _make_sc_segsum(n_pad, d, e):
    ec = e // NW
    nb = ec // G
    rows_pt = n_pad // NS
    mesh = plsc.VectorSubcoreMesh(core_axis_name="c", subcore_axis_name="s")

    @functools.partial(
        pl.kernel,
        out_type=(
            jax.ShapeDtypeStruct((n_pad, d), jnp.float32),
            jax.ShapeDtypeStruct((n_pad, d), jnp.float32),
        ),
        mesh=mesh,
        scratch_types=[
            pltpu.VMEM((2, G), jnp.int32),
            pltpu.VMEM((2, G), jnp.int32),
            pltpu.VMEM((G, d), jnp.float32),
            pltpu.VMEM((G, d), jnp.float32),
            pltpu.VMEM_SHARED((n_pad, d), jnp.float32),
            pltpu.SemaphoreType.DMA,
            pltpu.SemaphoreType.DMA,
            pltpu.SemaphoreType.DMA,
            pltpu.SemaphoreType.DMA,
        ],
    )
    def sc_segsum(sd_hbm, hw_hbm, zrows_hbm, p0_hbm, p1_hbm,
                  idxa_v, idxb_v, rowsa_v, rowsb_v, acc_sh,
                  semia, semib, semga, semgb):
        cid = lax.axis_index("c")
        sid = lax.axis_index("s")
        wid = cid * NS + sid
        gb0 = wid * nb

        pltpu.sync_copy(zrows_hbm, acc_sh.at[pl.ds(sid * rows_pt, rows_pt)])
        plsc.subcore_barrier()

        def wait_idx(dst_ref, sem):
            pltpu.make_async_copy(sd_hbm.at[0], dst_ref, sem).wait()

        def wait_rows(dst_ref, sem):
            pltpu.make_async_copy(hw_hbm.at[pl.ds(0, G)], dst_ref, sem).wait()

        pltpu.sync_copy(sd_hbm.at[gb0], idxa_v)
        pltpu.async_copy(hw_hbm.at[idxa_v.at[0]], rowsa_v, semga)
        pltpu.async_copy(sd_hbm.at[gb0 + 1], idxb_v, semib)

        def pair(t, carry):
            ba = gb0 + 2 * t
            wait_idx(idxb_v, semib)
            pltpu.async_copy(hw_hbm.at[idxb_v.at[0]], rowsb_v, semgb)
            wait_rows(rowsa_v, semga)
            pltpu.sync_copy(rowsa_v, acc_sh.at[idxa_v.at[1]], add=True)
            pltpu.async_copy(sd_hbm.at[ba + 2], idxa_v, semia)
            wait_idx(idxa_v, semia)
            pltpu.async_copy(hw_hbm.at[idxa_v.at[0]], rowsa_v, semga)
            wait_rows(rowsb_v, semgb)
            pltpu.sync_copy(rowsb_v, acc_sh.at[idxb_v.at[1]], add=True)
            pltpu.async_copy(sd_hbm.at[ba + 3], idxb_v, semib)
            return carry
        lax.fori_loop(0, (nb - 1) // 2, pair, 0)

        wait_rows(rowsa_v, semga)
        pltpu.sync_copy(rowsa_v, acc_sh.at[idxa_v.at[1]], add=True)
        wait_idx(idxb_v, semib)
        plsc.subcore_barrier()

        @pl.when(cid == 0)
        def _():
            pltpu.sync_copy(acc_sh.at[pl.ds(sid * rows_pt, rows_pt)],
                            p0_hbm.at[pl.ds(sid * rows_pt, rows_pt)])

        @pl.when(cid == 1)
        def _():
            pltpu.sync_copy(acc_sh.at[pl.ds(sid * rows_pt, rows_pt)],
                            p1_hbm.at[pl.ds(sid * rows_pt, rows_pt)])

    return sc_segsum


def _make_sc_loss(n_pad, d, m_pad, m_real):
    mc = m_pad // NW
    mesh = plsc.VectorSubcoreMesh(core_axis_name="c", subcore_axis_name="s")

    @functools.partial(
        pl.kernel,
        out_type=jax.ShapeDtypeStruct((NW * L,), jnp.float32),
        mesh=mesh,
        scratch_types=[
            pltpu.VMEM((G,), jnp.int32),
            pltpu.VMEM((G, d), jnp.float32),
            pltpu.VMEM((G, d), jnp.float32),
            pltpu.VMEM((G, d), jnp.float32),
            pltpu.VMEM((d,), jnp.float32),
            pltpu.VMEM((L,), jnp.float32),
            pltpu.SemaphoreType.DMA,
        ],
    )
    def sc_loss(p0_hbm, p1_hbm, x_hbm, mi_hbm, b_hbm, out_hbm,
                mi_v, r0_v, r1_v, rx_v, b_v, acc_v, sem):
        cid = lax.axis_index("c")
        sid = lax.axis_index("s")
        wid = cid * NS + sid

        pltpu.sync_copy(b_hbm, b_v)
        acc_v[...] = jnp.zeros((L,), jnp.float32)

        for j in range(mc // G):
            off = wid * mc + j * G
            pltpu.sync_copy(mi_hbm.at[pl.ds(off, G)], mi_v)
            pltpu.async_copy(p0_hbm.at[mi_v], r0_v, sem).wait()
            pltpu.async_copy(p1_hbm.at[mi_v], r1_v, sem).wait()
            pltpu.async_copy(x_hbm.at[mi_v], rx_v, sem).wait()

            def row(r, carry):
                vs = jnp.where(off + r < m_real, jnp.float32(1.0),
                               jnp.float32(0.0))
                valid = jnp.full((L,), vs)
                s = jnp.zeros((L,), jnp.float32)
                for c in range(d // L):
                    cs = pl.ds(c * L, L)
                    dv = (r0_v[r, cs] + r1_v[r, cs] + b_v[cs]) - rx_v[r, cs]
                    s = s + dv * dv
                acc_v[...] = acc_v[...] + valid * s
                return carry
            lax.fori_loop(0, G, row, 0)

        pltpu.sync_copy(acc_v, out_hbm.at[pl.ds(wid * L, L)])

    return sc_loss


def kernel(x, h, edge_index, mask_nodes, W_dec, b_dec, inference=False):
    n, d = x.shape
    e = edge_index.shape[1]
    m = mask_nodes.shape[0]
    mc = -(-m // NW)
    mc = -(-mc // G) * G
    m_pad = mc * NW
    n_pad = -(-n // (NS * 8)) * (NS * 8)

    src = edge_index[0].astype(jnp.int32)
    dst = edge_index[1].astype(jnp.int32)
    sd = jnp.concatenate([src.reshape(-1, 1, G), dst.reshape(-1, 1, G)],
                         axis=1)
    sd = jnp.concatenate([sd, jnp.zeros((1, 2, G), jnp.int32)], axis=0)
    mi = jnp.zeros((m_pad,), jnp.int32).at[:m].set(mask_nodes.astype(jnp.int32))
    zrows = jnp.zeros((n_pad // NS, d), jnp.float32)

    hw = _matmul(h, W_dec)
    p0, p1 = _make_sc_segsum(n_pad, d, e)(sd, hw, zrows)
    lane_sums = _make_sc_loss(n_pad, d, m_pad, m)(p0, p1, x, mi, b_dec)
    loss = jnp.sum(lane_sums) / jnp.float32(m * d)
    return jnp.where(inference, jnp.float32(0.0), loss)


# speedup vs baseline: 9.0078x; 1.0583x over previous
import functools
import jax
import jax.numpy as jnp
from jax import lax
from jax.experimental import pallas as pl
from jax.experimental.pallas import tpu as pltpu
from jax.experimental.pallas import tpu_sc as plsc

NC = 2
NS = 16
NW = NC * NS
G = 80
L = 16


def _mm_body(h_ref, w_ref, o_ref):
    o_ref[...] = jnp.dot(h_ref[...], w_ref[...],
                         preferred_element_type=jnp.float32)


def _matmul(h, w):
    n, d = h.shape
    blk = 1000
    return pl.pallas_call(
        _mm_body,
        grid=(n // blk,),
        in_specs=[
            pl.BlockSpec((blk, d), lambda i: (i, 0)),
            pl.BlockSpec((d, d), lambda i: (0, 0)),
        ],
        out_specs=pl.BlockSpec((blk, d), lambda i: (i, 0)),
        out_shape=jax.ShapeDtypeStruct((n, d), jnp.float32),
    )(h, w)


def _make_sc_segsum(n_pad, d, e):
    ec = e // NW
    nb = ec // G
    rows_pt = n_pad // NS
    mesh = plsc.VectorSubcoreMesh(core_axis_name="c", subcore_axis_name="s")

    @functools.partial(
        pl.kernel,
        out_type=(
            jax.ShapeDtypeStruct((n_pad, d), jnp.float32),
            jax.ShapeDtypeStruct((n_pad, d), jnp.float32),
        ),
        mesh=mesh,
        scratch_types=[
            pltpu.VMEM((2, G), jnp.int32),
            pltpu.VMEM((2, G), jnp.int32),
            pltpu.VMEM((G, d), jnp.float32),
            pltpu.VMEM((G, d), jnp.float32),
            pltpu.VMEM_SHARED((n_pad, d), jnp.float32),
            pltpu.SemaphoreType.DMA,
            pltpu.SemaphoreType.DMA,
            pltpu.SemaphoreType.DMA,
            pltpu.SemaphoreType.DMA,
        ],
    )
    def sc_segsum(sd_hbm, hw_hbm, zrows_hbm, p0_hbm, p1_hbm,
                  idxa_v, idxb_v, rowsa_v, rowsb_v, acc_sh,
                  semia, semib, semga, semgb):
        cid = lax.axis_index("c")
        sid = lax.axis_index("s")
        wid = cid * NS + sid
        gb0 = wid * nb

        pltpu.sync_copy(zrows_hbm, acc_sh.at[pl.ds(sid * rows_pt, rows_pt)])
        plsc.subcore_barrier()

        def wait_idx(dst_ref, sem):
            pltpu.make_async_copy(sd_hbm.at[0], dst_ref, sem).wait()

        def wait_rows(dst_ref, sem):
            pltpu.make_async_copy(hw_hbm.at[pl.ds(0, G)], dst_ref, sem).wait()

        pltpu.sync_copy(sd_hbm.at[gb0], idxa_v)
        pltpu.async_copy(hw_hbm.at[idxa_v.at[0]], rowsa_v, semga)
        pltpu.async_copy(sd_hbm.at[gb0 + 1], idxb_v, semib)

        def pair(t, carry):
            ba = gb0 + 2 * t
            wait_idx(idxb_v, semib)
            pltpu.async_copy(hw_hbm.at[idxb_v.at[0]], rowsb_v, semgb)
            wait_rows(rowsa_v, semga)
            pltpu.sync_copy(rowsa_v, acc_sh.at[idxa_v.at[1]], add=True)
            pltpu.async_copy(sd_hbm.at[ba + 2], idxa_v, semia)
            wait_idx(idxa_v, semia)
            pltpu.async_copy(hw_hbm.at[idxa_v.at[0]], rowsa_v, semga)
            wait_rows(rowsb_v, semgb)
            pltpu.sync_copy(rowsb_v, acc_sh.at[idxb_v.at[1]], add=True)
            pltpu.async_copy(sd_hbm.at[ba + 3], idxb_v, semib)
            return carry
        lax.fori_loop(0, (nb - 1) // 2, pair, 0)

        wait_rows(rowsa_v, semga)
        pltpu.sync_copy(rowsa_v, acc_sh.at[idxa_v.at[1]], add=True)
        wait_idx(idxb_v, semib)
        plsc.subcore_barrier()

        @pl.when(cid == 0)
        def _():
            pltpu.sync_copy(acc_sh.at[pl.ds(sid * rows_pt, rows_pt)],
                            p0_hbm.at[pl.ds(sid * rows_pt, rows_pt)])

        @pl.when(cid == 1)
        def _():
            pltpu.sync_copy(acc_sh.at[pl.ds(sid * rows_pt, rows_pt)],
                            p1_hbm.at[pl.ds(sid * rows_pt, rows_pt)])

    return sc_segsum


def _make_sc_loss(n_pad, d, m_pad, m_real):
    mc = m_pad // NW
    mesh = plsc.VectorSubcoreMesh(core_axis_name="c", subcore_axis_name="s")

    @functools.partial(
        pl.kernel,
        out_type=jax.ShapeDtypeStruct((NW * L,), jnp.float32),
        mesh=mesh,
        scratch_types=[
            [pltpu.VMEM((G,), jnp.int32) for _ in range(2)],
            [pltpu.VMEM((G, d), jnp.float32) for _ in range(6)],
            pltpu.VMEM((d,), jnp.float32),
            pltpu.VMEM((L,), jnp.float32),
            [pltpu.SemaphoreType.DMA for _ in range(2)],
        ],
    )
    def sc_loss(p0_hbm, p1_hbm, x_hbm, mi_hbm, b_hbm, out_hbm,
                mi_v, rows_v, b_v, acc_v, sems):
        cid = lax.axis_index("c")
        sid = lax.axis_index("s")
        wid = cid * NS + sid
        nchunk = mc // G

        pltpu.sync_copy(b_hbm, b_v)
        acc_v[...] = jnp.zeros((L,), jnp.float32)

        for j in range(nchunk):
            off = wid * mc + j * G
            pltpu.sync_copy(mi_hbm.at[pl.ds(off, G)], mi_v[j])
            pltpu.async_copy(p0_hbm.at[mi_v[j]], rows_v[3 * j + 0], sems[j])
            pltpu.async_copy(p1_hbm.at[mi_v[j]], rows_v[3 * j + 1], sems[j])
            pltpu.async_copy(x_hbm.at[mi_v[j]], rows_v[3 * j + 2], sems[j])

        for j in range(nchunk):
            off = wid * mc + j * G
            for _ in range(3):
                pltpu.make_async_copy(
                    p0_hbm.at[pl.ds(0, G)], rows_v[3 * j], sems[j]).wait()
            r0_v, r1_v, rx_v = rows_v[3 * j], rows_v[3 * j + 1], rows_v[3 * j + 2]

            def row(r, carry):
                vs = jnp.where(off + r < m_real, jnp.float32(1.0),
                               jnp.float32(0.0))
                valid = jnp.full((L,), vs)
                s = jnp.zeros((L,), jnp.float32)
                for c in range(d // L):
                    cs = pl.ds(c * L, L)
                    dv = (r0_v[r, cs] + r1_v[r, cs] + b_v[cs]) - rx_v[r, cs]
                    s = s + dv * dv
                acc_v[...] = acc_v[...] + valid * s
                return carry
            lax.fori_loop(0, G, row, 0)

        pltpu.sync_copy(acc_v, out_hbm.at[pl.ds(wid * L, L)])

    return sc_loss


def kernel(x, h, edge_index, mask_nodes, W_dec, b_dec, inference=False):
    n, d = x.shape
    e = edge_index.shape[1]
    m = mask_nodes.shape[0]
    mc = -(-m // NW)
    mc = -(-mc // G) * G
    m_pad = mc * NW
    n_pad = -(-n // (NS * 8)) * (NS * 8)

    src = edge_index[0].astype(jnp.int32)
    dst = edge_index[1].astype(jnp.int32)
    sd = jnp.concatenate([src.reshape(-1, 1, G), dst.reshape(-1, 1, G)],
                         axis=1)
    sd = jnp.concatenate([sd, jnp.zeros((1, 2, G), jnp.int32)], axis=0)
    mi = jnp.zeros((m_pad,), jnp.int32).at[:m].set(mask_nodes.astype(jnp.int32))
    zrows = jnp.zeros((n_pad // NS, d), jnp.float32)

    hw = _matmul(h, W_dec)
    p0, p1 = _make_sc_segsum(n_pad, d, e)(sd, hw, zrows)
    lane_sums = _make_sc_loss(n_pad, d, m_pad, m)(p0, p1, x, mi, b_dec)
    loss = jnp.sum(lane_sums) / jnp.float32(m * d)
    return jnp.where(inference, jnp.float32(0.0), loss)


# speedup vs baseline: 9.0643x; 1.0063x over previous
import functools
import jax
import jax.numpy as jnp
from jax import lax
from jax.experimental import pallas as pl
from jax.experimental.pallas import tpu as pltpu
from jax.experimental.pallas import tpu_sc as plsc

NC = 2
NS = 16
NW = NC * NS
G = 80
NSLOT = 4
L = 16


def _mm_body(h_ref, w_ref, o_ref):
    o_ref[...] = jnp.dot(h_ref[...], w_ref[...],
                         preferred_element_type=jnp.float32)


def _matmul(h, w):
    n, d = h.shape
    blk = 1000
    return pl.pallas_call(
        _mm_body,
        grid=(n // blk,),
        in_specs=[
            pl.BlockSpec((blk, d), lambda i: (i, 0)),
            pl.BlockSpec((d, d), lambda i: (0, 0)),
        ],
        out_specs=pl.BlockSpec((blk, d), lambda i: (i, 0)),
        out_shape=jax.ShapeDtypeStruct((n, d), jnp.float32),
    )(h, w)


def _make_sc_segsum(n_pad, d, e):
    ec = e // NW
    nb = ec // G
    rows_pt = n_pad // NS
    mesh = plsc.VectorSubcoreMesh(core_axis_name="c", subcore_axis_name="s")

    @functools.partial(
        pl.kernel,
        out_type=(
            jax.ShapeDtypeStruct((n_pad, d), jnp.float32),
            jax.ShapeDtypeStruct((n_pad, d), jnp.float32),
        ),
        mesh=mesh,
        scratch_types=[
            [pltpu.VMEM((2, G), jnp.int32) for _ in range(NSLOT)],
            [pltpu.VMEM((G, d), jnp.float32) for _ in range(NSLOT)],
            pltpu.VMEM_SHARED((n_pad, d), jnp.float32),
            [pltpu.SemaphoreType.DMA for _ in range(NSLOT)],
            [pltpu.SemaphoreType.DMA for _ in range(NSLOT)],
        ],
    )
    def sc_segsum(sd_hbm, hw_hbm, zrows_hbm, p0_hbm, p1_hbm,
                  idx_v, rows_v, acc_sh, semi, semg):
        cid = lax.axis_index("c")
        sid = lax.axis_index("s")
        wid = cid * NS + sid
        gb0 = wid * nb

        pltpu.sync_copy(zrows_hbm, acc_sh.at[pl.ds(sid * rows_pt, rows_pt)])
        plsc.subcore_barrier()

        def wait_idx(s):
            pltpu.make_async_copy(sd_hbm.at[0], idx_v[s], semi[s]).wait()

        def wait_rows(s):
            pltpu.make_async_copy(hw_hbm.at[pl.ds(0, G)], rows_v[s],
                                  semg[s]).wait()

        for s in range(NSLOT):
            pltpu.sync_copy(sd_hbm.at[gb0 + s], idx_v[s])
            pltpu.async_copy(hw_hbm.at[idx_v[s].at[0]], rows_v[s], semg[s])

        def group(t, carry):
            base = gb0 + NSLOT * t
            for s in range(NSLOT):
                wait_rows(s)
                pltpu.sync_copy(rows_v[s], acc_sh.at[idx_v[s].at[1]],
                                add=True)
                pltpu.async_copy(sd_hbm.at[base + s + NSLOT], idx_v[s],
                                 semi[s])
                wait_idx(s)
                pltpu.async_copy(hw_hbm.at[idx_v[s].at[0]], rows_v[s],
                                 semg[s])
            return carry
        ngroup = nb // NSLOT
        lax.fori_loop(0, ngroup, group, 0)

        for s in range(NSLOT):
            wait_rows(s)
            if ngroup * NSLOT + s < nb:
                pltpu.sync_copy(rows_v[s], acc_sh.at[idx_v[s].at[1]],
                                add=True)
        plsc.subcore_barrier()

        @pl.when(cid == 0)
        def _():
            pltpu.sync_copy(acc_sh.at[pl.ds(sid * rows_pt, rows_pt)],
                            p0_hbm.at[pl.ds(sid * rows_pt, rows_pt)])

        @pl.when(cid == 1)
        def _():
            pltpu.sync_copy(acc_sh.at[pl.ds(sid * rows_pt, rows_pt)],
                            p1_hbm.at[pl.ds(sid * rows_pt, rows_pt)])

    return sc_segsum


def _make_sc_loss(n_pad, d, m_pad, m_real):
    mc = m_pad // NW
    mesh = plsc.VectorSubcoreMesh(core_axis_name="c", subcore_axis_name="s")

    @functools.partial(
        pl.kernel,
        out_type=jax.ShapeDtypeStruct((NW * L,), jnp.float32),
        mesh=mesh,
        scratch_types=[
            [pltpu.VMEM((G,), jnp.int32) for _ in range(2)],
            [pltpu.VMEM((G, d), jnp.float32) for _ in range(6)],
            pltpu.VMEM((d,), jnp.float32),
            pltpu.VMEM((L,), jnp.float32),
            [pltpu.SemaphoreType.DMA for _ in range(2)],
        ],
    )
    def sc_loss(p0_hbm, p1_hbm, x_hbm, mi_hbm, b_hbm, out_hbm,
                mi_v, rows_v, b_v, acc_v, sems):
        cid = lax.axis_index("c")
        sid = lax.axis_index("s")
        wid = cid * NS + sid
        nchunk = mc // G

        pltpu.sync_copy(b_hbm, b_v)
        acc_v[...] = jnp.zeros((L,), jnp.float32)

        for j in range(nchunk):
            off = wid * mc + j * G
            pltpu.sync_copy(mi_hbm.at[pl.ds(off, G)], mi_v[j])
            pltpu.async_copy(p0_hbm.at[mi_v[j]], rows_v[3 * j + 0], sems[j])
            pltpu.async_copy(p1_hbm.at[mi_v[j]], rows_v[3 * j + 1], sems[j])
            pltpu.async_copy(x_hbm.at[mi_v[j]], rows_v[3 * j + 2], sems[j])

        for j in range(nchunk):
            off = wid * mc + j * G
            for _ in range(3):
                pltpu.make_async_copy(
                    p0_hbm.at[pl.ds(0, G)], rows_v[3 * j], sems[j]).wait()
            r0_v, r1_v, rx_v = rows_v[3 * j], rows_v[3 * j + 1], rows_v[3 * j + 2]

            def row(r, carry):
                vs = jnp.where(off + r < m_real, jnp.float32(1.0),
                               jnp.float32(0.0))
                valid = jnp.full((L,), vs)
                s = jnp.zeros((L,), jnp.float32)
                for c in range(d // L):
                    cs = pl.ds(c * L, L)
                    dv = (r0_v[r, cs] + r1_v[r, cs] + b_v[cs]) - rx_v[r, cs]
                    s = s + dv * dv
                acc_v[...] = acc_v[...] + valid * s
                return carry
            lax.fori_loop(0, G, row, 0)

        pltpu.sync_copy(acc_v, out_hbm.at[pl.ds(wid * L, L)])

    return sc_loss


def kernel(x, h, edge_index, mask_nodes, W_dec, b_dec, inference=False):
    n, d = x.shape
    e = edge_index.shape[1]
    m = mask_nodes.shape[0]
    mc = -(-m // NW)
    mc = -(-mc // G) * G
    m_pad = mc * NW
    n_pad = -(-n // (NS * 8)) * (NS * 8)

    src = edge_index[0].astype(jnp.int32)
    dst = edge_index[1].astype(jnp.int32)
    sd = jnp.concatenate([src.reshape(-1, 1, G), dst.reshape(-1, 1, G)],
                         axis=1)
    sd = jnp.concatenate([sd, jnp.zeros((NSLOT, 2, G), jnp.int32)], axis=0)
    mi = jnp.zeros((m_pad,), jnp.int32).at[:m].set(mask_nodes.astype(jnp.int32))
    zrows = jnp.zeros((n_pad // NS, d), jnp.float32)

    hw = _matmul(h, W_dec)
    p0, p1 = _make_sc_segsum(n_pad, d, e)(sd, hw, zrows)
    lane_sums = _make_sc_loss(n_pad, d, m_pad, m)(p0, p1, x, mi, b_dec)
    loss = jnp.sum(lane_sums) / jnp.float32(m * d)
    return jnp.where(inference, jnp.float32(0.0), loss)


# speedup vs baseline: 9.2409x; 1.0195x over previous
import functools
import jax
import jax.numpy as jnp
from jax import lax
from jax.experimental import pallas as pl
from jax.experimental.pallas import tpu as pltpu
from jax.experimental.pallas import tpu_sc as plsc

NC = 2
NS = 16
NW = NC * NS
G = 80
NSLOT = 4
L = 16


def _mm_body(h_ref, w_ref, o_ref):
    o_ref[...] = jnp.dot(h_ref[...], w_ref[...],
                         preferred_element_type=jnp.float32)


def _matmul(h, w):
    n, d = h.shape
    blk = 1000
    return pl.pallas_call(
        _mm_body,
        grid=(n // blk,),
        in_specs=[
            pl.BlockSpec((blk, d), lambda i: (i, 0)),
            pl.BlockSpec((d, d), lambda i: (0, 0)),
        ],
        out_specs=pl.BlockSpec((blk, d), lambda i: (i, 0)),
        out_shape=jax.ShapeDtypeStruct((n, d), jnp.float32),
    )(h, w)


def _make_sc_segsum(n_pad, d, e):
    ec = e // NW
    nb = ec // G
    rows_pt = n_pad // NS
    mesh = plsc.VectorSubcoreMesh(core_axis_name="c", subcore_axis_name="s")

    @functools.partial(
        pl.kernel,
        out_type=(
            jax.ShapeDtypeStruct((n_pad, d), jnp.float32),
            jax.ShapeDtypeStruct((n_pad, d), jnp.float32),
        ),
        mesh=mesh,
        scratch_types=[
            [pltpu.VMEM((2, G), jnp.int32) for _ in range(NSLOT)],
            [pltpu.VMEM((G, d), jnp.float32) for _ in range(NSLOT)],
            pltpu.VMEM_SHARED((n_pad, d), jnp.float32),
            [pltpu.SemaphoreType.DMA for _ in range(NSLOT)],
            [pltpu.SemaphoreType.DMA for _ in range(NSLOT)],
            [pltpu.SemaphoreType.DMA for _ in range(NSLOT)],
        ],
    )
    def sc_segsum(sd_hbm, hw_hbm, zrows_hbm, p0_hbm, p1_hbm,
                  idx_v, rows_v, acc_sh, semi, semg, sems):
        cid = lax.axis_index("c")
        sid = lax.axis_index("s")
        wid = cid * NS + sid
        gb0 = wid * nb

        pltpu.sync_copy(zrows_hbm, acc_sh.at[pl.ds(sid * rows_pt, rows_pt)])
        plsc.subcore_barrier()

        def wait_idx(s):
            pltpu.make_async_copy(sd_hbm.at[0], idx_v[s], semi[s]).wait()

        def wait_rows(s):
            pltpu.make_async_copy(hw_hbm.at[pl.ds(0, G)], rows_v[s],
                                  semg[s]).wait()

        def wait_scat(s):
            pltpu.make_async_copy(rows_v[s], acc_sh.at[pl.ds(0, G)],
                                  sems[s]).wait()

        for s in range(NSLOT):
            pltpu.sync_copy(sd_hbm.at[gb0 + s], idx_v[s])
            pltpu.async_copy(hw_hbm.at[idx_v[s].at[0]], rows_v[s], semg[s])

        def group(t, carry):
            base = gb0 + NSLOT * t
            for s in range(NSLOT):
                wait_rows(s)
                pltpu.async_copy(rows_v[s], acc_sh.at[idx_v[s].at[1]],
                                 sems[s], add=True)
            for s in range(NSLOT):
                wait_scat(s)
                pltpu.async_copy(sd_hbm.at[base + s + NSLOT], idx_v[s],
                                 semi[s])
            for s in range(NSLOT):
                wait_idx(s)
                pltpu.async_copy(hw_hbm.at[idx_v[s].at[0]], rows_v[s],
                                 semg[s])
            return carry
        ngroup = nb // NSLOT
        lax.fori_loop(0, ngroup, group, 0)

        for s in range(NSLOT):
            wait_rows(s)
            if ngroup * NSLOT + s < nb:
                pltpu.async_copy(rows_v[s], acc_sh.at[idx_v[s].at[1]],
                                 sems[s], add=True)
        for s in range(NSLOT):
            if ngroup * NSLOT + s < nb:
                wait_scat(s)
        plsc.subcore_barrier()

        @pl.when(cid == 0)
        def _():
            pltpu.sync_copy(acc_sh.at[pl.ds(sid * rows_pt, rows_pt)],
                            p0_hbm.at[pl.ds(sid * rows_pt, rows_pt)])

        @pl.when(cid == 1)
        def _():
            pltpu.sync_copy(acc_sh.at[pl.ds(sid * rows_pt, rows_pt)],
                            p1_hbm.at[pl.ds(sid * rows_pt, rows_pt)])

    return sc_segsum


def _make_sc_loss(n_pad, d, m_pad, m_real):
    mc = m_pad // NW
    mesh = plsc.VectorSubcoreMesh(core_axis_name="c", subcore_axis_name="s")

    @functools.partial(
        pl.kernel,
        out_type=jax.ShapeDtypeStruct((NW * L,), jnp.float32),
        mesh=mesh,
        scratch_types=[
            [pltpu.VMEM((G,), jnp.int32) for _ in range(2)],
            [pltpu.VMEM((G, d), jnp.float32) for _ in range(6)],
            pltpu.VMEM((d,), jnp.float32),
            pltpu.VMEM((L,), jnp.float32),
            [pltpu.SemaphoreType.DMA for _ in range(2)],
        ],
    )
    def sc_loss(p0_hbm, p1_hbm, x_hbm, mi_hbm, b_hbm, out_hbm,
                mi_v, rows_v, b_v, acc_v, sems):
        cid = lax.axis_index("c")
        sid = lax.axis_index("s")
        wid = cid * NS + sid
        nchunk = mc // G

        pltpu.sync_copy(b_hbm, b_v)
        acc_v[...] = jnp.zeros((L,), jnp.float32)

        for j in range(nchunk):
            off = wid * mc + j * G
            pltpu.sync_copy(mi_hbm.at[pl.ds(off, G)], mi_v[j])
            pltpu.async_copy(p0_hbm.at[mi_v[j]], rows_v[3 * j + 0], sems[j])
            pltpu.async_copy(p1_hbm.at[mi_v[j]], rows_v[3 * j + 1], sems[j])
            pltpu.async_copy(x_hbm.at[mi_v[j]], rows_v[3 * j + 2], sems[j])

        for j in range(nchunk):
            off = wid * mc + j * G
            for _ in range(3):
                pltpu.make_async_copy(
                    p0_hbm.at[pl.ds(0, G)], rows_v[3 * j], sems[j]).wait()
            r0_v, r1_v, rx_v = rows_v[3 * j], rows_v[3 * j + 1], rows_v[3 * j + 2]

            def row(r, carry):
                vs = jnp.where(off + r < m_real, jnp.float32(1.0),
                               jnp.float32(0.0))
                valid = jnp.full((L,), vs)
                s = jnp.zeros((L,), jnp.float32)
                for c in range(d // L):
                    cs = pl.ds(c * L, L)
                    dv = (r0_v[r, cs] + r1_v[r, cs] + b_v[cs]) - rx_v[r, cs]
                    s = s + dv * dv
                acc_v[...] = acc_v[...] + valid * s
                return carry
            lax.fori_loop(0, G, row, 0)

        pltpu.sync_copy(acc_v, out_hbm.at[pl.ds(wid * L, L)])

    return sc_loss


def kernel(x, h, edge_index, mask_nodes, W_dec, b_dec, inference=False):
    n, d = x.shape
    e = edge_index.shape[1]
    m = mask_nodes.shape[0]
    mc = -(-m // NW)
    mc = -(-mc // G) * G
    m_pad = mc * NW
    n_pad = -(-n // (NS * 8)) * (NS * 8)

    src = edge_index[0].astype(jnp.int32)
    dst = edge_index[1].astype(jnp.int32)
    sd = jnp.concatenate([src.reshape(-1, 1, G), dst.reshape(-1, 1, G)],
                         axis=1)
    sd = jnp.concatenate([sd, jnp.zeros((NSLOT, 2, G), jnp.int32)], axis=0)
    mi = jnp.zeros((m_pad,), jnp.int32).at[:m].set(mask_nodes.astype(jnp.int32))
    zrows = jnp.zeros((n_pad // NS, d), jnp.float32)

    hw = _matmul(h, W_dec)
    p0, p1 = _make_sc_segsum(n_pad, d, e)(sd, hw, zrows)
    lane_sums = _make_sc_loss(n_pad, d, m_pad, m)(p0, p1, x, mi, b_dec)
    loss = jnp.sum(lane_sums) / jnp.float32(m * d)
    return jnp.where(inference, jnp.float32(0.0), loss)
